# baseline (speedup 1.0000x reference)
import functools

import jax
import jax.numpy as jnp
from jax import lax
from jax.experimental import pallas as pl
from jax.experimental.pallas import tpu as pltpu
from jax.experimental.pallas import tpu_sc as plsc

N_NODES = 10000
NP = 10240
EP = 327680
EPT = 10240
NCH = 80
CH = 128
RPT = NP // 16
DUMMY = N_NODES

_mesh = plsc.VectorSubcoreMesh(core_axis_name="c", subcore_axis_name="s")


@functools.partial(
    pl.kernel,
    mesh=_mesh,
    out_type=jax.ShapeDtypeStruct((2, NP), jnp.float32),
    scratch_types=[
        pltpu.VMEM((CH,), jnp.int32),
        pltpu.VMEM((CH,), jnp.int32),
        pltpu.VMEM((CH,), jnp.int32),
        pltpu.VMEM((CH,), jnp.int32),
        pltpu.VMEM((EPT,), jnp.int32),
        pltpu.VMEM((CH,), jnp.float32),
        pltpu.VMEM((RPT,), jnp.float32),
        pltpu.VMEM_SHARED((NP,), jnp.float32),
        pltpu.SemaphoreType.DMA,
        pltpu.SemaphoreType.DMA,
        pltpu.SemaphoreType.DMA,
        pltpu.SemaphoreType.DMA,
    ],
)
def _deg_kernel(colf_hbm, out_hbm, cb0, cb1, cb2, cb3, colv, ones_v, stage, acc,
                sm0, sm1, sm2, sm3):
    c = lax.axis_index("c")
    s = lax.axis_index("s")
    wid = c * 16 + s
    cbs = (cb0, cb1, cb2, cb3)
    sms = (sm0, sm1, sm2, sm3)

    def fill_ones(j, carry):
        ones_v[pl.ds(j * 16, 16)] = jnp.ones((16,), jnp.float32)
        return carry

    lax.fori_loop(0, CH // 16, fill_ones, 0)

    def fill_zero(j, carry):
        stage[pl.ds(j * 16, 16)] = jnp.zeros((16,), jnp.float32)
        return carry

    lax.fori_loop(0, RPT // 16, fill_zero, 0)
    pltpu.sync_copy(stage, acc.at[pl.ds(s * RPT, RPT)])
    plsc.subcore_barrier()

    pltpu.sync_copy(colf_hbm.at[pl.ds(wid * EPT, EPT)], colv)

    def unpack(g, cb):
        def cpy(j, cc):
            cb[pl.ds(j * 16, 16)] = colv[pl.ds(g * CH + j * 16, 16)]
            return cc

        lax.fori_loop(0, CH // 16, cpy, 0)

    for k in range(4):
        unpack(k, cbs[k])
        pltpu.async_copy(ones_v, acc.at[cbs[k]], sms[k], add=True)

    def body(t, carry):
        for k in range(4):
            g = 4 * t + k
            pltpu.make_async_copy(
                ones_v, acc.at[pl.ds(0, CH)], sms[k]
            ).wait()

            @pl.when(g + 4 < NCH)
            def _():
                unpack(g + 4, cbs[k])
                pltpu.async_copy(ones_v, acc.at[cbs[k]], sms[k], add=True)

        return carry

    lax.fori_loop(0, NCH // 4, body, 0)
    plsc.subcore_barrier()
    pltpu.sync_copy(acc.at[pl.ds(s * RPT, RPT)], stage)
    pltpu.sync_copy(stage, out_hbm.at[c, pl.ds(s * RPT, RPT)])


@functools.partial(
    pl.kernel,
    mesh=_mesh,
    out_type=jax.ShapeDtypeStruct((2, NP, 128), jnp.float32),
    scratch_types=[
        pltpu.VMEM((2 * CH,), jnp.int32),
        pltpu.VMEM((2 * CH,), jnp.int32),
        pltpu.VMEM((CH,), jnp.int32),
        pltpu.VMEM((CH,), jnp.int32),
        pltpu.VMEM((CH,), jnp.int32),
        pltpu.VMEM((CH,), jnp.int32),
        pltpu.VMEM((CH, 128), jnp.float32),
        pltpu.VMEM((CH, 128), jnp.float32),
        pltpu.VMEM_SHARED((NP, 128), jnp.float32),
        pltpu.SemaphoreType.DMA,
        pltpu.SemaphoreType.DMA,
        pltpu.SemaphoreType.DMA,
        pltpu.SemaphoreType.DMA,
        pltpu.SemaphoreType.DMA,
        pltpu.SemaphoreType.DMA,
    ],
)
def _scatter_kernel(y_hbm, rc_hbm, zeros_hbm, out_hbm,
                    rca, rcb, rowa, cola, rowb, colb, bufa, bufb, acc,
                    sema, semb, ssema, ssemb, rcsema, rcsemb):
    c = lax.axis_index("c")
    s = lax.axis_index("s")
    wid = c * 16 + s

    def zinit(i, carry):
        pltpu.sync_copy(zeros_hbm.at[pl.ds(s * RPT + i * CH, CH)], bufa)
        pltpu.sync_copy(bufa, acc.at[pl.ds(s * RPT + i * CH, CH)])
        return carry

    lax.fori_loop(0, RPT // CH, zinit, 0)
    plsc.subcore_barrier()

    rcbase = wid * EPT * 2

    def unpack(rc, rowx, colx):
        for j in range(CH // 16):
            rowx[pl.ds(j * 16, 16)] = rc[pl.ds(j * 16, 16)]
            colx[pl.ds(j * 16, 16)] = rc[pl.ds(CH + j * 16, 16)]

    pltpu.async_copy(rc_hbm.at[pl.ds(rcbase, 2 * CH)], rca, rcsema).wait()
    unpack(rca, rowa, cola)
    pltpu.async_copy(y_hbm.at[rowa], bufa, sema)
    pltpu.async_copy(rc_hbm.at[pl.ds(rcbase + 2 * CH, 2 * CH)], rcb, rcsemb)
    pltpu.make_async_copy(rc_hbm.at[pl.ds(0, 2 * CH)], rcb, rcsemb).wait()
    unpack(rcb, rowb, colb)
    pltpu.async_copy(y_hbm.at[rowb], bufb, semb)

    def body(t, carry):
        g0 = 2 * t
        pltpu.make_async_copy(y_hbm.at[pl.ds(0, CH)], bufa, sema).wait()
        pltpu.async_copy(bufa, acc.at[cola], ssema, add=True)

        @pl.when(g0 + 2 < NCH)
        def _():
            pltpu.async_copy(
                rc_hbm.at[pl.ds(rcbase + (g0 + 2) * 2 * CH, 2 * CH)],
                rca, rcsema,
            )

        pltpu.make_async_copy(y_hbm.at[pl.ds(0, CH)], bufb, semb).wait()
        pltpu.async_copy(bufb, acc.at[colb], ssemb, add=True)

        @pl.when(g0 + 3 < NCH)
        def _():
            pltpu.async_copy(
                rc_hbm.at[pl.ds(rcbase + (g0 + 3) * 2 * CH, 2 * CH)],
                rcb, rcsemb,
            )

        pltpu.make_async_copy(bufa, acc.at[pl.ds(0, CH)], ssema).wait()

        @pl.when(g0 + 2 < NCH)
        def _():
            pltpu.make_async_copy(
                rc_hbm.at[pl.ds(0, 2 * CH)], rca, rcsema
            ).wait()
            unpack(rca, rowa, cola)
            pltpu.async_copy(y_hbm.at[rowa], bufa, sema)

        pltpu.make_async_copy(bufb, acc.at[pl.ds(0, CH)], ssemb).wait()

        @pl.when(g0 + 3 < NCH)
        def _():
            pltpu.make_async_copy(
                rc_hbm.at[pl.ds(0, 2 * CH)], rcb, rcsemb
            ).wait()
            unpack(rcb, rowb, colb)
            pltpu.async_copy(y_hbm.at[rowb], bufb, semb)

        return carry

    lax.fori_loop(0, NCH // 2, body, 0)
    plsc.subcore_barrier()

    def rout(i, carry):
        pltpu.sync_copy(acc.at[pl.ds(s * RPT + i * CH, CH)], bufa)
        pltpu.sync_copy(bufa, out_hbm.at[c, pl.ds(s * RPT + i * CH, CH)])
        return carry

    lax.fori_loop(0, RPT // CH, rout, 0)


_BR = 1024


def _mm_scale_body(x_ref, w_ref, dinv_ref, o_ref):
    o_ref[...] = dinv_ref[...] * jnp.dot(
        x_ref[...], w_ref[...], preferred_element_type=jnp.float32
    )


def _mid_body(sa_ref, sb_ref, y_ref, dinv_ref, b_ref, w_ref, o_ref):
    h = dinv_ref[...] * (sa_ref[...] + sb_ref[...] + y_ref[...]) + b_ref[...]
    h = jnp.maximum(h, 0.0)
    o_ref[...] = dinv_ref[...] * jnp.dot(
        h, w_ref[...], preferred_element_type=jnp.float32
    )


def _final_body(sa_ref, sb_ref, y_ref, dinv_ref, b_ref, o_ref):
    o_ref[...] = (
        dinv_ref[...] * (sa_ref[...] + sb_ref[...] + y_ref[...]) + b_ref[...]
    )


def _row_spec():
    return pl.BlockSpec((_BR, 128), lambda i: (i, 0))


def _full_spec():
    return pl.BlockSpec((128, 128), lambda i: (0, 0))


def _dinv_spec():
    return pl.BlockSpec((_BR, 1), lambda i: (i, 0))


def _bias_spec():
    return pl.BlockSpec((1, 128), lambda i: (0, 0))


def _mm_scale(x, w, dinv2):
    return pl.pallas_call(
        _mm_scale_body,
        grid=(NP // _BR,),
        in_specs=[_row_spec(), _full_spec(), _dinv_spec()],
        out_specs=_row_spec(),
        out_shape=jax.ShapeDtypeStruct((NP, 128), jnp.float32),
    )(x, w, dinv2)


def _mid(sa, sb, y, dinv2, b, w):
    return pl.pallas_call(
        _mid_body,
        grid=(NP // _BR,),
        in_specs=[_row_spec(), _row_spec(), _row_spec(), _dinv_spec(),
                  _bias_spec(), _full_spec()],
        out_specs=_row_spec(),
        out_shape=jax.ShapeDtypeStruct((NP, 128), jnp.float32),
    )(sa, sb, y, dinv2, b, w)


def _final(sa, sb, y, dinv2, b):
    return pl.pallas_call(
        _final_body,
        grid=(NP // _BR,),
        in_specs=[_row_spec(), _row_spec(), _row_spec(), _dinv_spec(),
                  _bias_spec()],
        out_specs=_row_spec(),
        out_shape=jax.ShapeDtypeStruct((NP, 128), jnp.float32),
    )(sa, sb, y, dinv2, b)


def kernel(x, edge_index, W1, b1, W2, b2):
    E = edge_index.shape[1]
    row = edge_index[0].astype(jnp.int32)
    col = edge_index[1].astype(jnp.int32)
    npad = EP - E
    pad_src = jnp.arange(npad, dtype=jnp.int32) % N_NODES
    pad_dst = DUMMY + (jnp.arange(npad, dtype=jnp.int32) % (NP - N_NODES))
    row_p = jnp.concatenate([row, pad_src])
    col_p = jnp.concatenate([col, pad_dst])
    rc = jnp.stack(
        [row_p.reshape(EP // CH, CH), col_p.reshape(EP // CH, CH)], axis=1
    ).reshape(2 * EP)
    x_p = jnp.concatenate(
        [x, jnp.zeros((NP - N_NODES, 128), jnp.float32)], axis=0
    )
    zeros_hbm = jnp.zeros((NP, 128), jnp.float32)
    b1r = b1.reshape(1, 128)
    b2r = b2.reshape(1, 128)

    deg_p = _deg_kernel(col_p)
    deg = deg_p[0] + deg_p[1] + 1.0
    dinv2 = lax.rsqrt(deg)[:, None]

    y1 = _mm_scale(x_p, W1, dinv2)
    s1 = _scatter_kernel(y1, rc, zeros_hbm)
    y2 = _mid(s1[0], s1[1], y1, dinv2, b1r, W2)
    s2 = _scatter_kernel(y2, rc, zeros_hbm)
    out = _final(s2[0], s2[1], y2, dinv2, b2r)
    return out[:N_NODES]


# speedup vs baseline: 1.1222x; 1.1222x over previous
import functools

import jax
import jax.numpy as jnp
from jax import lax
from jax.experimental import pallas as pl
from jax.experimental.pallas import tpu as pltpu
from jax.experimental.pallas import tpu_sc as plsc

N_NODES = 10000
NP = 10240
EP = 327680
EPT = 10240
NCH = 80
CH = 128
RPT = NP // 16
DUMMY = N_NODES

_mesh = plsc.VectorSubcoreMesh(core_axis_name="c", subcore_axis_name="s")


@functools.partial(
    pl.kernel,
    mesh=_mesh,
    out_type=jax.ShapeDtypeStruct((2, NP), jnp.float32),
    scratch_types=[
        pltpu.VMEM((CH,), jnp.int32),
        pltpu.VMEM((CH,), jnp.int32),
        pltpu.VMEM((CH,), jnp.int32),
        pltpu.VMEM((CH,), jnp.int32),
        pltpu.VMEM((EPT,), jnp.int32),
        pltpu.VMEM((CH,), jnp.float32),
        pltpu.VMEM((RPT,), jnp.float32),
        pltpu.VMEM_SHARED((NP,), jnp.float32),
        pltpu.SemaphoreType.DMA,
        pltpu.SemaphoreType.DMA,
        pltpu.SemaphoreType.DMA,
        pltpu.SemaphoreType.DMA,
    ],
)
def _deg_kernel(colf_hbm, out_hbm, cb0, cb1, cb2, cb3, colv, ones_v, stage, acc,
                sm0, sm1, sm2, sm3):
    c = lax.axis_index("c")
    s = lax.axis_index("s")
    wid = c * 16 + s
    cbs = (cb0, cb1, cb2, cb3)
    sms = (sm0, sm1, sm2, sm3)

    def fill_ones(j, carry):
        ones_v[pl.ds(j * 16, 16)] = jnp.ones((16,), jnp.float32)
        return carry

    lax.fori_loop(0, CH // 16, fill_ones, 0)

    def fill_zero(j, carry):
        stage[pl.ds(j * 16, 16)] = jnp.zeros((16,), jnp.float32)
        return carry

    lax.fori_loop(0, RPT // 16, fill_zero, 0)
    pltpu.sync_copy(stage, acc.at[pl.ds(s * RPT, RPT)])
    plsc.subcore_barrier()

    pltpu.sync_copy(colf_hbm.at[pl.ds(wid * EPT, EPT)], colv)

    def unpack(g, cb):
        def cpy(j, cc):
            cb[pl.ds(j * 16, 16)] = colv[pl.ds(g * CH + j * 16, 16)]
            return cc

        lax.fori_loop(0, CH // 16, cpy, 0)

    for k in range(4):
        unpack(k, cbs[k])
        pltpu.async_copy(ones_v, acc.at[cbs[k]], sms[k], add=True)

    def body(t, carry):
        for k in range(4):
            g = 4 * t + k
            pltpu.make_async_copy(
                ones_v, acc.at[pl.ds(0, CH)], sms[k]
            ).wait()

            @pl.when(g + 4 < NCH)
            def _():
                unpack(g + 4, cbs[k])
                pltpu.async_copy(ones_v, acc.at[cbs[k]], sms[k], add=True)

        return carry

    lax.fori_loop(0, NCH // 4, body, 0)
    plsc.subcore_barrier()
    pltpu.sync_copy(acc.at[pl.ds(s * RPT, RPT)], stage)
    pltpu.sync_copy(stage, out_hbm.at[c, pl.ds(s * RPT, RPT)])


@functools.partial(
    pl.kernel,
    mesh=_mesh,
    out_type=jax.ShapeDtypeStruct((2, NP, 128), jnp.float32),
    scratch_types=[
        pltpu.VMEM((CH,), jnp.int32),
        pltpu.VMEM((CH,), jnp.int32),
        pltpu.VMEM((CH,), jnp.int32),
        pltpu.VMEM((CH,), jnp.int32),
        pltpu.VMEM((CH, 128), jnp.float32),
        pltpu.VMEM((CH, 128), jnp.float32),
        pltpu.VMEM_SHARED((NP, 128), jnp.float32),
        pltpu.SemaphoreType.DMA,
        pltpu.SemaphoreType.DMA,
        pltpu.SemaphoreType.DMA,
        pltpu.SemaphoreType.DMA,
    ],
)
def _scatter_kernel(y_hbm, rowf_hbm, colf_hbm, zeros_hbm, out_hbm,
                    rowa, cola, rowb, colb, bufa, bufb, acc,
                    sema, semb, ssema, ssemb):
    c = lax.axis_index("c")
    s = lax.axis_index("s")
    wid = c * 16 + s

    def zinit(i, carry):
        pltpu.sync_copy(zeros_hbm.at[pl.ds(s * RPT + i * CH, CH)], bufa)
        pltpu.sync_copy(bufa, acc.at[pl.ds(s * RPT + i * CH, CH)])
        return carry

    lax.fori_loop(0, RPT // CH, zinit, 0)
    plsc.subcore_barrier()

    base0 = wid * EPT
    pltpu.sync_copy(rowf_hbm.at[pl.ds(base0, CH)], rowa)
    pltpu.sync_copy(colf_hbm.at[pl.ds(base0, CH)], cola)
    pltpu.async_copy(y_hbm.at[rowa], bufa, sema)
    pltpu.sync_copy(rowf_hbm.at[pl.ds(base0 + CH, CH)], rowb)
    pltpu.sync_copy(colf_hbm.at[pl.ds(base0 + CH, CH)], colb)

    def body(t, carry):
        g0 = 2 * t
        pltpu.async_copy(y_hbm.at[rowb], bufb, semb)
        pltpu.make_async_copy(y_hbm.at[pl.ds(0, CH)], bufa, sema).wait()
        pltpu.async_copy(bufa, acc.at[cola], ssema, add=True)

        @pl.when(g0 + 2 < NCH)
        def _():
            pltpu.sync_copy(rowf_hbm.at[pl.ds(base0 + (g0 + 2) * CH, CH)], rowa)
            pltpu.make_async_copy(bufa, acc.at[pl.ds(0, CH)], ssema).wait()
            pltpu.sync_copy(colf_hbm.at[pl.ds(base0 + (g0 + 2) * CH, CH)], cola)
            pltpu.async_copy(y_hbm.at[rowa], bufa, sema)

        @pl.when(g0 + 2 >= NCH)
        def _():
            pltpu.make_async_copy(bufa, acc.at[pl.ds(0, CH)], ssema).wait()

        pltpu.make_async_copy(y_hbm.at[pl.ds(0, CH)], bufb, semb).wait()
        pltpu.async_copy(bufb, acc.at[colb], ssemb, add=True)

        @pl.when(g0 + 3 < NCH)
        def _():
            pltpu.sync_copy(rowf_hbm.at[pl.ds(base0 + (g0 + 3) * CH, CH)], rowb)
            pltpu.make_async_copy(bufb, acc.at[pl.ds(0, CH)], ssemb).wait()
            pltpu.sync_copy(colf_hbm.at[pl.ds(base0 + (g0 + 3) * CH, CH)], colb)

        @pl.when(g0 + 3 >= NCH)
        def _():
            pltpu.make_async_copy(bufb, acc.at[pl.ds(0, CH)], ssemb).wait()

        return carry

    lax.fori_loop(0, NCH // 2, body, 0)
    plsc.subcore_barrier()

    def rout(i, carry):
        pltpu.sync_copy(acc.at[pl.ds(s * RPT + i * CH, CH)], bufa)
        pltpu.sync_copy(bufa, out_hbm.at[c, pl.ds(s * RPT + i * CH, CH)])
        return carry

    lax.fori_loop(0, RPT // CH, rout, 0)


_BR = 1024


def _mm_scale_body(x_ref, w_ref, dinv_ref, o_ref):
    o_ref[...] = dinv_ref[...] * jnp.dot(
        x_ref[...], w_ref[...], preferred_element_type=jnp.float32
    )


def _mid_body(sa_ref, sb_ref, y_ref, dinv_ref, b_ref, w_ref, o_ref):
    h = dinv_ref[...] * (sa_ref[...] + sb_ref[...] + y_ref[...]) + b_ref[...]
    h = jnp.maximum(h, 0.0)
    o_ref[...] = dinv_ref[...] * jnp.dot(
        h, w_ref[...], preferred_element_type=jnp.float32
    )


def _final_body(sa_ref, sb_ref, y_ref, dinv_ref, b_ref, o_ref):
    o_ref[...] = (
        dinv_ref[...] * (sa_ref[...] + sb_ref[...] + y_ref[...]) + b_ref[...]
    )


def _row_spec():
    return pl.BlockSpec((_BR, 128), lambda i: (i, 0))


def _full_spec():
    return pl.BlockSpec((128, 128), lambda i: (0, 0))


def _dinv_spec():
    return pl.BlockSpec((_BR, 1), lambda i: (i, 0))


def _bias_spec():
    return pl.BlockSpec((1, 128), lambda i: (0, 0))


def _mm_scale(x, w, dinv2):
    return pl.pallas_call(
        _mm_scale_body,
        grid=(NP // _BR,),
        in_specs=[_row_spec(), _full_spec(), _dinv_spec()],
        out_specs=_row_spec(),
        out_shape=jax.ShapeDtypeStruct((NP, 128), jnp.float32),
    )(x, w, dinv2)


def _mid(sa, sb, y, dinv2, b, w):
    return pl.pallas_call(
        _mid_body,
        grid=(NP // _BR,),
        in_specs=[_row_spec(), _row_spec(), _row_spec(), _dinv_spec(),
                  _bias_spec(), _full_spec()],
        out_specs=_row_spec(),
        out_shape=jax.ShapeDtypeStruct((NP, 128), jnp.float32),
    )(sa, sb, y, dinv2, b, w)


def _final(sa, sb, y, dinv2, b):
    return pl.pallas_call(
        _final_body,
        grid=(NP // _BR,),
        in_specs=[_row_spec(), _row_spec(), _row_spec(), _dinv_spec(),
                  _bias_spec()],
        out_specs=_row_spec(),
        out_shape=jax.ShapeDtypeStruct((N_NODES, 128), jnp.float32),
    )(sa, sb, y, dinv2, b)


def kernel(x, edge_index, W1, b1, W2, b2):
    E = edge_index.shape[1]
    row = edge_index[0].astype(jnp.int32)
    col = edge_index[1].astype(jnp.int32)
    npad = EP - E
    pad_src = jnp.arange(npad, dtype=jnp.int32) % N_NODES
    pad_dst = DUMMY + (jnp.arange(npad, dtype=jnp.int32) % (NP - N_NODES))
    row_p = jnp.concatenate([row, pad_src])
    col_p = jnp.concatenate([col, pad_dst])
    zeros_hbm = jnp.zeros((NP, 128), jnp.float32)
    b1r = b1.reshape(1, 128)
    b2r = b2.reshape(1, 128)

    deg_p = _deg_kernel(col_p)
    deg = deg_p[0] + deg_p[1] + 1.0
    dinv2 = lax.rsqrt(deg)[:, None]

    y1 = _mm_scale(x, W1, dinv2)
    s1 = _scatter_kernel(y1, row_p, col_p, zeros_hbm)
    y2 = _mid(s1[0], s1[1], y1, dinv2, b1r, W2)
    s2 = _scatter_kernel(y2, row_p, col_p, zeros_hbm)
    return _final(s2[0], s2[1], y2, dinv2, b2r)


# speedup vs baseline: 1.1659x; 1.0390x over previous
import functools

import jax
import jax.numpy as jnp
from jax import lax
from jax.experimental import pallas as pl
from jax.experimental.pallas import tpu as pltpu
from jax.experimental.pallas import tpu_sc as plsc

N_NODES = 10000
NP = 10240
EP = 327680
EPT = 10240
NCH = 80
CH = 128
RPT = NP // 16
DUMMY = N_NODES

_mesh = plsc.VectorSubcoreMesh(core_axis_name="c", subcore_axis_name="s")


@functools.partial(
    pl.kernel,
    mesh=_mesh,
    out_type=jax.ShapeDtypeStruct((2, NP), jnp.float32),
    scratch_types=[
        pltpu.VMEM((CH,), jnp.int32),
        pltpu.VMEM((CH,), jnp.int32),
        pltpu.VMEM((CH,), jnp.int32),
        pltpu.VMEM((CH,), jnp.int32),
        pltpu.VMEM((EPT,), jnp.int32),
        pltpu.VMEM((CH,), jnp.float32),
        pltpu.VMEM((RPT,), jnp.float32),
        pltpu.VMEM_SHARED((NP,), jnp.float32),
        pltpu.SemaphoreType.DMA,
        pltpu.SemaphoreType.DMA,
        pltpu.SemaphoreType.DMA,
        pltpu.SemaphoreType.DMA,
    ],
)
def _deg_kernel(colf_hbm, out_hbm, cb0, cb1, cb2, cb3, colv, ones_v, stage, acc,
                sm0, sm1, sm2, sm3):
    c = lax.axis_index("c")
    s = lax.axis_index("s")
    wid = c * 16 + s
    cbs = (cb0, cb1, cb2, cb3)
    sms = (sm0, sm1, sm2, sm3)

    def fill_ones(j, carry):
        ones_v[pl.ds(j * 16, 16)] = jnp.ones((16,), jnp.float32)
        return carry

    lax.fori_loop(0, CH // 16, fill_ones, 0)

    def fill_zero(j, carry):
        stage[pl.ds(j * 16, 16)] = jnp.zeros((16,), jnp.float32)
        return carry

    lax.fori_loop(0, RPT // 16, fill_zero, 0)
    pltpu.sync_copy(stage, acc.at[pl.ds(s * RPT, RPT)])
    plsc.subcore_barrier()

    pltpu.sync_copy(colf_hbm.at[pl.ds(wid * EPT, EPT)], colv)

    def unpack(g, cb):
        def cpy(j, cc):
            cb[pl.ds(j * 16, 16)] = colv[pl.ds(g * CH + j * 16, 16)]
            return cc

        lax.fori_loop(0, CH // 16, cpy, 0)

    for k in range(4):
        unpack(k, cbs[k])
        pltpu.async_copy(ones_v, acc.at[cbs[k]], sms[k], add=True)

    def body(t, carry):
        for k in range(4):
            g = 4 * t + k
            pltpu.make_async_copy(
                ones_v, acc.at[pl.ds(0, CH)], sms[k]
            ).wait()

            @pl.when(g + 4 < NCH)
            def _():
                unpack(g + 4, cbs[k])
                pltpu.async_copy(ones_v, acc.at[cbs[k]], sms[k], add=True)

        return carry

    lax.fori_loop(0, NCH // 4, body, 0)
    plsc.subcore_barrier()
    pltpu.sync_copy(acc.at[pl.ds(s * RPT, RPT)], stage)
    pltpu.sync_copy(stage, out_hbm.at[c, pl.ds(s * RPT, RPT)])


@functools.partial(
    pl.kernel,
    mesh=_mesh,
    out_type=jax.ShapeDtypeStruct((2, NP, 128), jnp.float32),
    scratch_types=[
        pltpu.VMEM((CH,), jnp.int32),
        pltpu.VMEM((CH,), jnp.int32),
        pltpu.VMEM((CH,), jnp.int32),
        pltpu.VMEM((CH,), jnp.int32),
        pltpu.VMEM((CH, 128), jnp.float32),
        pltpu.VMEM((CH, 128), jnp.float32),
        pltpu.VMEM_SHARED((NP, 128), jnp.float32),
        pltpu.SemaphoreType.DMA,
        pltpu.SemaphoreType.DMA,
        pltpu.SemaphoreType.DMA,
        pltpu.SemaphoreType.DMA,
    ],
)
def _scatter_kernel(y_hbm, rowf_hbm, colf_hbm, out_hbm,
                    rowa, cola, rowb, colb, bufa, bufb, acc,
                    sema, semb, ssema, ssemb):
    c = lax.axis_index("c")
    s = lax.axis_index("s")
    wid = c * 16 + s

    def fill_zero(i, carry):
        bufa[i // 8, pl.ds((i % 8) * 16, 16)] = jnp.zeros((16,), jnp.float32)
        return carry

    lax.fori_loop(0, CH * 8, fill_zero, 0)
    for i in range(RPT // CH):
        pltpu.async_copy(bufa, acc.at[pl.ds(s * RPT + i * CH, CH)], ssema)
    for i in range(RPT // CH):
        pltpu.make_async_copy(bufa, acc.at[pl.ds(0, CH)], ssema).wait()
    plsc.subcore_barrier()

    base0 = wid * EPT
    pltpu.sync_copy(rowf_hbm.at[pl.ds(base0, CH)], rowa)
    pltpu.sync_copy(colf_hbm.at[pl.ds(base0, CH)], cola)
    pltpu.async_copy(y_hbm.at[rowa], bufa, sema)
    pltpu.sync_copy(rowf_hbm.at[pl.ds(base0 + CH, CH)], rowb)
    pltpu.sync_copy(colf_hbm.at[pl.ds(base0 + CH, CH)], colb)

    def body(t, carry):
        g0 = 2 * t
        pltpu.async_copy(y_hbm.at[rowb], bufb, semb)
        pltpu.make_async_copy(y_hbm.at[pl.ds(0, CH)], bufa, sema).wait()
        pltpu.async_copy(bufa, acc.at[cola], ssema, add=True)

        @pl.when(g0 + 2 < NCH)
        def _():
            pltpu.sync_copy(rowf_hbm.at[pl.ds(base0 + (g0 + 2) * CH, CH)], rowa)
            pltpu.make_async_copy(bufa, acc.at[pl.ds(0, CH)], ssema).wait()
            pltpu.sync_copy(colf_hbm.at[pl.ds(base0 + (g0 + 2) * CH, CH)], cola)
            pltpu.async_copy(y_hbm.at[rowa], bufa, sema)

        @pl.when(g0 + 2 >= NCH)
        def _():
            pltpu.make_async_copy(bufa, acc.at[pl.ds(0, CH)], ssema).wait()

        pltpu.make_async_copy(y_hbm.at[pl.ds(0, CH)], bufb, semb).wait()
        pltpu.async_copy(bufb, acc.at[colb], ssemb, add=True)

        @pl.when(g0 + 3 < NCH)
        def _():
            pltpu.sync_copy(rowf_hbm.at[pl.ds(base0 + (g0 + 3) * CH, CH)], rowb)
            pltpu.make_async_copy(bufb, acc.at[pl.ds(0, CH)], ssemb).wait()
            pltpu.sync_copy(colf_hbm.at[pl.ds(base0 + (g0 + 3) * CH, CH)], colb)

        @pl.when(g0 + 3 >= NCH)
        def _():
            pltpu.make_async_copy(bufb, acc.at[pl.ds(0, CH)], ssemb).wait()

        return carry

    lax.fori_loop(0, NCH // 2, body, 0)
    plsc.subcore_barrier()

    nro = RPT // CH
    bufs = (bufa, bufb)
    rsems = (sema, semb)
    wsems = (ssema, ssemb)
    pltpu.async_copy(acc.at[pl.ds(s * RPT, CH)], bufa, sema)
    for i in range(nro):
        k = i % 2
        if i + 1 < nro:
            if i >= 1:
                pltpu.make_async_copy(
                    bufs[1 - k], out_hbm.at[0, pl.ds(0, CH)], wsems[1 - k]
                ).wait()
            pltpu.async_copy(
                acc.at[pl.ds(s * RPT + (i + 1) * CH, CH)],
                bufs[1 - k], rsems[1 - k],
            )
        pltpu.make_async_copy(
            acc.at[pl.ds(0, CH)], bufs[k], rsems[k]
        ).wait()
        pltpu.async_copy(
            bufs[k], out_hbm.at[c, pl.ds(s * RPT + i * CH, CH)], wsems[k]
        )
    pltpu.make_async_copy(bufs[(nro - 1) % 2], out_hbm.at[0, pl.ds(0, CH)],
                          wsems[(nro - 1) % 2]).wait()
    pltpu.make_async_copy(bufs[(nro - 2) % 2], out_hbm.at[0, pl.ds(0, CH)],
                          wsems[(nro - 2) % 2]).wait()


_BR = 1024


def _mm_scale_body(x_ref, w_ref, dinv_ref, o_ref):
    o_ref[...] = dinv_ref[...] * jnp.dot(
        x_ref[...], w_ref[...], preferred_element_type=jnp.float32
    )


def _mid_body(sa_ref, sb_ref, y_ref, dinv_ref, b_ref, w_ref, o_ref):
    h = dinv_ref[...] * (sa_ref[...] + sb_ref[...] + y_ref[...]) + b_ref[...]
    h = jnp.maximum(h, 0.0)
    o_ref[...] = dinv_ref[...] * jnp.dot(
        h, w_ref[...], preferred_element_type=jnp.float32
    )


def _final_body(sa_ref, sb_ref, y_ref, dinv_ref, b_ref, o_ref):
    o_ref[...] = (
        dinv_ref[...] * (sa_ref[...] + sb_ref[...] + y_ref[...]) + b_ref[...]
    )


def _row_spec():
    return pl.BlockSpec((_BR, 128), lambda i: (i, 0))


def _full_spec():
    return pl.BlockSpec((128, 128), lambda i: (0, 0))


def _dinv_spec():
    return pl.BlockSpec((_BR, 1), lambda i: (i, 0))


def _bias_spec():
    return pl.BlockSpec((1, 128), lambda i: (0, 0))


def _mm_scale(x, w, dinv2):
    return pl.pallas_call(
        _mm_scale_body,
        grid=(NP // _BR,),
        in_specs=[_row_spec(), _full_spec(), _dinv_spec()],
        out_specs=_row_spec(),
        out_shape=jax.ShapeDtypeStruct((NP, 128), jnp.float32),
    )(x, w, dinv2)


def _mid(sa, sb, y, dinv2, b, w):
    return pl.pallas_call(
        _mid_body,
        grid=(NP // _BR,),
        in_specs=[_row_spec(), _row_spec(), _row_spec(), _dinv_spec(),
                  _bias_spec(), _full_spec()],
        out_specs=_row_spec(),
        out_shape=jax.ShapeDtypeStruct((NP, 128), jnp.float32),
    )(sa, sb, y, dinv2, b, w)


def _final(sa, sb, y, dinv2, b):
    return pl.pallas_call(
        _final_body,
        grid=(NP // _BR,),
        in_specs=[_row_spec(), _row_spec(), _row_spec(), _dinv_spec(),
                  _bias_spec()],
        out_specs=_row_spec(),
        out_shape=jax.ShapeDtypeStruct((N_NODES, 128), jnp.float32),
    )(sa, sb, y, dinv2, b)


def kernel(x, edge_index, W1, b1, W2, b2):
    E = edge_index.shape[1]
    row = edge_index[0].astype(jnp.int32)
    col = edge_index[1].astype(jnp.int32)
    npad = EP - E
    pad_src = jnp.arange(npad, dtype=jnp.int32) % N_NODES
    pad_dst = DUMMY + (jnp.arange(npad, dtype=jnp.int32) % (NP - N_NODES))
    row_p = jnp.concatenate([row, pad_src])
    col_p = jnp.concatenate([col, pad_dst])
    b1r = b1.reshape(1, 128)
    b2r = b2.reshape(1, 128)

    deg_p = _deg_kernel(col_p)
    deg = deg_p[0] + deg_p[1] + 1.0
    dinv2 = lax.rsqrt(deg)[:, None]

    y1 = _mm_scale(x, W1, dinv2)
    s1 = _scatter_kernel(y1, row_p, col_p)
    y2 = _mid(s1[0], s1[1], y1, dinv2, b1r, W2)
    s2 = _scatter_kernel(y2, row_p, col_p)
    return _final(s2[0], s2[1], y2, dinv2, b2r)
